# Initial kernel scaffold; baseline (speedup 1.0000x reference)
import jax
import jax.numpy as jnp
from jax.experimental import pallas as pl


def kernel(x, edge_index, edge_type, W):
    raise NotImplementedError("write your pallas kernel here")


# speedup vs baseline: 6.9821x; 6.9821x over previous
import functools

import jax
import jax.numpy as jnp
import numpy as np
from jax import lax
from jax.experimental import pallas as pl
from jax.experimental.pallas import tpu as pltpu
from jax.experimental.pallas import tpu_sc as plsc

N = 10000
E = 320000
C = 128
H = C // 2
T = 7
NC, NS = 2, 16
EPT = E // NS
CHUNK = 80
NCHUNK = EPT // CHUNK
NROWC = N // CHUNK
SCALE = 1.0 / (T * np.sqrt(float(C)))

_f32 = jnp.float32


def _proj_body(x_ref, w_ref, y_ref):
    x = x_ref[...]
    y_ref[0, 0] = (
        jnp.dot(x, w_ref[0, :, :H], preferred_element_type=_f32) * SCALE
    )
    y_ref[1, 0] = (
        jnp.dot(x, w_ref[0, :, H:], preferred_element_type=_f32) * SCALE
    )


_BN = 2000


def _project(x, w3):
    return pl.pallas_call(
        _proj_body,
        grid=(N // _BN, T),
        in_specs=[
            pl.BlockSpec((_BN, C), lambda nb, t: (nb, 0)),
            pl.BlockSpec((1, C, C), lambda nb, t: (t, 0, 0)),
        ],
        out_specs=pl.BlockSpec((NC, 1, _BN, H), lambda nb, t: (0, t, nb, 0)),
        out_shape=jax.ShapeDtypeStruct((NC, T, N, H), _f32),
    )(x, w3)


def _gidx_body(col_ref, typ_ref, g_ref):
    g_ref[...] = typ_ref[...] * N + col_ref[...]


def _gather_index(col2d, typ2d):
    return pl.pallas_call(
        _gidx_body,
        out_shape=jax.ShapeDtypeStruct(col2d.shape, jnp.int32),
    )(col2d, typ2d)


_mesh = plsc.VectorSubcoreMesh(core_axis_name="c", subcore_axis_name="s")


@functools.partial(
    pl.kernel,
    out_type=jax.ShapeDtypeStruct((NC, N, H), _f32),
    mesh=_mesh,
    scratch_types=[
        pltpu.VMEM((NCHUNK, CHUNK), jnp.int32),
        pltpu.VMEM((NCHUNK, CHUNK), jnp.int32),
        pltpu.VMEM((CHUNK, H), _f32),
        pltpu.VMEM((CHUNK, H), _f32),
        pltpu.VMEM_SHARED((N, H), _f32),
        pltpu.SemaphoreType.DMA,
        pltpu.SemaphoreType.DMA,
    ],
    compiler_params=pltpu.CompilerParams(use_tc_tiling_on_sc=False),
)
def _sc_aggregate(y_hbm, gidx_hbm, rows_hbm, out_hbm,
                  gidx_v, rows_v, gb0, gb1, acc, sem0, sem1):
    cid = lax.axis_index("c")
    sid = lax.axis_index("s")
    yh = y_hbm.at[cid]

    pltpu.sync_copy(gidx_hbm.at[sid], gidx_v)
    pltpu.sync_copy(rows_hbm.at[sid], rows_v)

    zeros16 = jnp.zeros((16,), _f32)

    def _zrow(i, carry):
        for j in range(H // 16):
            gb0[i, pl.ds(j * 16, 16)] = zeros16
        return carry

    lax.fori_loop(0, CHUNK, _zrow, 0)
    for i in range(pl.cdiv(NROWC, NS)):
        zc = i * NS + sid

        @pl.when(zc < NROWC)
        def _zero_chunk():
            off = pl.multiple_of(zc * CHUNK, CHUNK)
            pltpu.sync_copy(gb0, acc.at[pl.ds(off, CHUNK)])

    plsc.subcore_barrier()

    pltpu.async_copy(yh.at[gidx_v.at[0]], gb0, sem0)
    pltpu.async_copy(yh.at[gidx_v.at[1]], gb1, sem1)

    def _body(j, carry):
        c0 = 2 * j
        c1 = c0 + 1
        pltpu.make_async_copy(yh.at[gidx_v.at[c0]], gb0, sem0).wait()
        pltpu.sync_copy(gb0, acc.at[rows_v.at[c0]], add=True)

        @pl.when(c0 + 2 < NCHUNK)
        def _pref0():
            pltpu.async_copy(yh.at[gidx_v.at[c0 + 2]], gb0, sem0)

        pltpu.make_async_copy(yh.at[gidx_v.at[c1]], gb1, sem1).wait()
        pltpu.sync_copy(gb1, acc.at[rows_v.at[c1]], add=True)

        @pl.when(c1 + 2 < NCHUNK)
        def _pref1():
            pltpu.async_copy(yh.at[gidx_v.at[c1 + 2]], gb1, sem1)

        return carry

    lax.fori_loop(0, NCHUNK // 2, _body, 0)

    plsc.subcore_barrier()
    for i in range(pl.cdiv(NROWC, NS)):
        wc = i * NS + sid

        @pl.when(wc < NROWC)
        def _wb_chunk():
            off = pl.multiple_of(wc * CHUNK, CHUNK)
            pltpu.sync_copy(acc.at[pl.ds(off, CHUNK)],
                            out_hbm.at[cid, pl.ds(off, CHUNK)])


def _merge_body(p_ref, o_ref):
    o_ref[:, :H] = p_ref[0]
    o_ref[:, H:] = p_ref[1]


def _merge(partials):
    return pl.pallas_call(
        _merge_body,
        grid=(N // _BN,),
        in_specs=[pl.BlockSpec((NC, _BN, H), lambda i: (0, i, 0))],
        out_specs=pl.BlockSpec((_BN, C), lambda i: (i, 0)),
        out_shape=jax.ShapeDtypeStruct((N, C), _f32),
    )(partials)


def kernel(x, edge_index, edge_type, W):
    row = edge_index[0]
    col = edge_index[1]
    w3 = W.reshape(T, C, C)

    y = _project(x, w3)
    g2d = _gather_index(
        col.reshape(E // C, C), edge_type.reshape(E // C, C)
    )
    gidx3 = g2d.reshape(NS, NCHUNK, CHUNK)
    rows3 = row.reshape(NS, NCHUNK, CHUNK)

    partials = _sc_aggregate(y.reshape(NC, T * N, H), gidx3, rows3)
    return _merge(partials)


# speedup vs baseline: 9.2781x; 1.3289x over previous
import functools

import jax
import jax.numpy as jnp
import numpy as np
from jax import lax
from jax.experimental import pallas as pl
from jax.experimental.pallas import tpu as pltpu
from jax.experimental.pallas import tpu_sc as plsc

N = 10000
E = 320000
C = 128
T = 7
NC, NS = 2, 16
NW = NC * NS
EPW = E // NW
CHUNK = 40
NCHUNK = EPW // CHUNK
NROWC = N // CHUNK
SCALE = 1.0 / (T * np.sqrt(float(C)))

_f32 = jnp.float32


def _proj_body(x_ref, w_ref, y_ref):
    y_ref[0] = (
        jnp.dot(x_ref[...], w_ref[0], preferred_element_type=_f32) * SCALE
    )


_BN = 2000


def _project(x, w3):
    return pl.pallas_call(
        _proj_body,
        grid=(N // _BN, T),
        in_specs=[
            pl.BlockSpec((_BN, C), lambda nb, t: (nb, 0)),
            pl.BlockSpec((1, C, C), lambda nb, t: (t, 0, 0)),
        ],
        out_specs=pl.BlockSpec((1, _BN, C), lambda nb, t: (t, nb, 0)),
        out_shape=jax.ShapeDtypeStruct((T, N, C), _f32),
    )(x, w3)


def _gidx_body(ei_ref, typ_ref, g_ref, r_ref):
    g_ref[...] = typ_ref[...] * N + ei_ref[1]
    r_ref[...] = ei_ref[0]


def _edge_lists(edge_index, edge_type):
    e2 = E // C
    return pl.pallas_call(
        _gidx_body,
        out_shape=(
            jax.ShapeDtypeStruct((e2, C), jnp.int32),
            jax.ShapeDtypeStruct((e2, C), jnp.int32),
        ),
    )(edge_index.reshape(2, e2, C), edge_type.reshape(e2, C))


_mesh = plsc.VectorSubcoreMesh(core_axis_name="c", subcore_axis_name="s")


@functools.partial(
    pl.kernel,
    out_type=jax.ShapeDtypeStruct((NC, N, C), _f32),
    mesh=_mesh,
    scratch_types=[
        pltpu.VMEM((NCHUNK, CHUNK), jnp.int32),
        pltpu.VMEM((NCHUNK, CHUNK), jnp.int32),
        pltpu.VMEM((CHUNK, C), _f32),
        pltpu.VMEM((CHUNK, C), _f32),
        pltpu.VMEM_SHARED((N, C), _f32),
        pltpu.SemaphoreType.DMA,
        pltpu.SemaphoreType.DMA,
    ],
    compiler_params=pltpu.CompilerParams(use_tc_tiling_on_sc=False),
)
def _sc_aggregate(y_hbm, gidx_hbm, rows_hbm, out_hbm,
                  gidx_v, rows_v, gb0, gb1, acc, sem0, sem1):
    cid = lax.axis_index("c")
    sid = lax.axis_index("s")
    wid = sid * NC + cid

    pltpu.sync_copy(gidx_hbm.at[wid], gidx_v)
    pltpu.sync_copy(rows_hbm.at[wid], rows_v)

    zeros16 = jnp.zeros((16,), _f32)

    def _zrow(i, carry):
        for j in range(C // 16):
            gb0[i, pl.ds(j * 16, 16)] = zeros16
        return carry

    lax.fori_loop(0, CHUNK, _zrow, 0)
    for i in range(pl.cdiv(NROWC, NS)):
        zc = i * NS + sid

        @pl.when(zc < NROWC)
        def _zero_chunk():
            off = pl.multiple_of(zc * CHUNK, CHUNK)
            pltpu.sync_copy(gb0, acc.at[pl.ds(off, CHUNK)])

    plsc.subcore_barrier()

    pltpu.async_copy(y_hbm.at[gidx_v.at[0]], gb0, sem0)

    def _body(j, carry):
        c0 = 2 * j
        c1 = c0 + 1
        pltpu.async_copy(y_hbm.at[gidx_v.at[c1]], gb1, sem1)
        pltpu.make_async_copy(y_hbm.at[gidx_v.at[c0]], gb0, sem0).wait()
        pltpu.sync_copy(gb0, acc.at[rows_v.at[c0]], add=True)
        pltpu.async_copy(y_hbm.at[gidx_v.at[c0 + 2]], gb0, sem0)
        pltpu.make_async_copy(y_hbm.at[gidx_v.at[c1]], gb1, sem1).wait()
        pltpu.sync_copy(gb1, acc.at[rows_v.at[c1]], add=True)
        return carry

    lax.fori_loop(0, NCHUNK // 2 - 1, _body, 0)
    pltpu.async_copy(y_hbm.at[gidx_v.at[NCHUNK - 1]], gb1, sem1)
    pltpu.make_async_copy(y_hbm.at[gidx_v.at[NCHUNK - 2]], gb0, sem0).wait()
    pltpu.sync_copy(gb0, acc.at[rows_v.at[NCHUNK - 2]], add=True)
    pltpu.make_async_copy(y_hbm.at[gidx_v.at[NCHUNK - 1]], gb1, sem1).wait()
    pltpu.sync_copy(gb1, acc.at[rows_v.at[NCHUNK - 1]], add=True)

    plsc.subcore_barrier()
    for i in range(pl.cdiv(NROWC, NS)):
        wc = i * NS + sid

        @pl.when(wc < NROWC)
        def _wb_chunk():
            off = pl.multiple_of(wc * CHUNK, CHUNK)
            pltpu.sync_copy(acc.at[pl.ds(off, CHUNK)],
                            out_hbm.at[cid, pl.ds(off, CHUNK)])


def _add_body(p_ref, o_ref):
    o_ref[...] = p_ref[0] + p_ref[1]


def _final_add(partials):
    return pl.pallas_call(
        _add_body,
        grid=(N // _BN,),
        in_specs=[pl.BlockSpec((NC, _BN, C), lambda i: (0, i, 0))],
        out_specs=pl.BlockSpec((_BN, C), lambda i: (i, 0)),
        out_shape=jax.ShapeDtypeStruct((N, C), _f32),
    )(partials)


def kernel(x, edge_index, edge_type, W):
    w3 = W.reshape(T, C, C)
    y = _project(x, w3)
    g2d, r2d = _edge_lists(edge_index, edge_type)
    gidx3 = g2d.reshape(NW, NCHUNK, CHUNK)
    rows3 = r2d.reshape(NW, NCHUNK, CHUNK)
    partials = _sc_aggregate(y.reshape(T * N, C), gidx3, rows3)
    return _final_add(partials)


# speedup vs baseline: 11.7043x; 1.2615x over previous
import functools

import jax
import jax.numpy as jnp
import numpy as np
from jax import lax
from jax.experimental import pallas as pl
from jax.experimental.pallas import tpu as pltpu
from jax.experimental.pallas import tpu_sc as plsc

N = 10000
E = 320000
C = 128
T = 7
NC, NS = 2, 16
NW = NC * NS
EPW = E // NW
CHUNK = 80
NCHUNK = EPW // CHUNK
NROWC = N // CHUNK
SCALE = 1.0 / (T * np.sqrt(float(C)))

_f32 = jnp.float32


def _proj_body(x_ref, w_ref, y_ref):
    y_ref[0] = (
        jnp.dot(x_ref[...], w_ref[0], preferred_element_type=_f32) * SCALE
    )


_BN = 2000


def _project(x, w3):
    return pl.pallas_call(
        _proj_body,
        grid=(N // _BN, T),
        in_specs=[
            pl.BlockSpec((_BN, C), lambda nb, t: (nb, 0)),
            pl.BlockSpec((1, C, C), lambda nb, t: (t, 0, 0)),
        ],
        out_specs=pl.BlockSpec((1, _BN, C), lambda nb, t: (t, nb, 0)),
        out_shape=jax.ShapeDtypeStruct((T, N, C), _f32),
    )(x, w3)


def _gidx_body(ei_ref, typ_ref, g_ref, r_ref):
    g_ref[...] = typ_ref[...] * N + ei_ref[1]
    r_ref[...] = ei_ref[0]


def _edge_lists(edge_index, edge_type):
    e2 = E // C
    return pl.pallas_call(
        _gidx_body,
        out_shape=(
            jax.ShapeDtypeStruct((e2, C), jnp.int32),
            jax.ShapeDtypeStruct((e2, C), jnp.int32),
        ),
    )(edge_index.reshape(2, e2, C), edge_type.reshape(e2, C))


_mesh = plsc.VectorSubcoreMesh(core_axis_name="c", subcore_axis_name="s")


@functools.partial(
    pl.kernel,
    out_type=jax.ShapeDtypeStruct((NC, N, C), _f32),
    mesh=_mesh,
    scratch_types=[
        pltpu.VMEM((NCHUNK, CHUNK), jnp.int32),
        pltpu.VMEM((1, CHUNK), jnp.int32),
        pltpu.VMEM((1, CHUNK), jnp.int32),
        pltpu.VMEM((CHUNK, C), _f32),
        pltpu.VMEM((CHUNK, C), _f32),
        pltpu.VMEM_SHARED((N, C), _f32),
        pltpu.SemaphoreType.DMA,
        pltpu.SemaphoreType.DMA,
    ],
    compiler_params=pltpu.CompilerParams(use_tc_tiling_on_sc=False),
)
def _sc_aggregate(y_hbm, gidx_hbm, rows_hbm, out_hbm,
                  gidx_v, rb0, rb1, gb0, gb1, acc, sem0, sem1):
    cid = lax.axis_index("c")
    sid = lax.axis_index("s")
    wid = sid * NC + cid
    rows_h = rows_hbm.at[wid]

    pltpu.sync_copy(gidx_hbm.at[wid], gidx_v)

    zeros16 = jnp.zeros((16,), _f32)

    def _zrow(i, carry):
        for j in range(C // 16):
            gb0[i, pl.ds(j * 16, 16)] = zeros16
        return carry

    lax.fori_loop(0, CHUNK, _zrow, 0)
    for i in range(pl.cdiv(NROWC, NS)):
        zc = i * NS + sid

        @pl.when(zc < NROWC)
        def _zero_chunk():
            off = pl.multiple_of(zc * CHUNK, CHUNK)
            pltpu.sync_copy(gb0, acc.at[pl.ds(off, CHUNK)])

    plsc.subcore_barrier()

    def _start0(c):
        pltpu.async_copy(y_hbm.at[gidx_v.at[c]], gb0, sem0)
        pltpu.async_copy(rows_h.at[pl.ds(c, 1)], rb0, sem0)

    def _finish0(c):
        pltpu.make_async_copy(y_hbm.at[gidx_v.at[c]], gb0, sem0).wait()
        pltpu.make_async_copy(rows_h.at[pl.ds(c, 1)], rb0, sem0).wait()
        pltpu.sync_copy(gb0, acc.at[rb0.at[0]], add=True)

    def _start1(c):
        pltpu.async_copy(y_hbm.at[gidx_v.at[c]], gb1, sem1)
        pltpu.async_copy(rows_h.at[pl.ds(c, 1)], rb1, sem1)

    def _finish1(c):
        pltpu.make_async_copy(y_hbm.at[gidx_v.at[c]], gb1, sem1).wait()
        pltpu.make_async_copy(rows_h.at[pl.ds(c, 1)], rb1, sem1).wait()
        pltpu.sync_copy(gb1, acc.at[rb1.at[0]], add=True)

    _start0(0)

    def _body(j, carry):
        c0 = 2 * j
        _start1(c0 + 1)
        _finish0(c0)
        _start0(c0 + 2)
        _finish1(c0 + 1)
        return carry

    lax.fori_loop(0, NCHUNK // 2, _body, 0)
    _finish0(NCHUNK - 1)

    plsc.subcore_barrier()
    for i in range(pl.cdiv(NROWC, NS)):
        wc = i * NS + sid

        @pl.when(wc < NROWC)
        def _wb_chunk():
            off = pl.multiple_of(wc * CHUNK, CHUNK)
            pltpu.sync_copy(acc.at[pl.ds(off, CHUNK)],
                            out_hbm.at[cid, pl.ds(off, CHUNK)])


def _add_body(p_ref, o_ref):
    o_ref[...] = p_ref[0] + p_ref[1]


def _final_add(partials):
    return pl.pallas_call(
        _add_body,
        grid=(N // _BN,),
        in_specs=[pl.BlockSpec((NC, _BN, C), lambda i: (0, i, 0))],
        out_specs=pl.BlockSpec((_BN, C), lambda i: (i, 0)),
        out_shape=jax.ShapeDtypeStruct((N, C), _f32),
    )(partials)


def kernel(x, edge_index, edge_type, W):
    w3 = W.reshape(T, C, C)
    y = _project(x, w3)
    g2d, r2d = _edge_lists(edge_index, edge_type)
    gidx3 = g2d.reshape(NW, NCHUNK, CHUNK)
    rows3 = r2d.reshape(NW, NCHUNK, CHUNK)
    partials = _sc_aggregate(y.reshape(T * N, C), gidx3, rows3)
    return _final_add(partials)


# speedup vs baseline: 14.8316x; 1.2672x over previous
import functools

import jax
import jax.numpy as jnp
import numpy as np
from jax import lax
from jax.experimental import pallas as pl
from jax.experimental.pallas import tpu as pltpu
from jax.experimental.pallas import tpu_sc as plsc

N = 10000
E = 320000
C = 128
T = 7
NC, NS = 2, 16
NW = NC * NS
EPW = E // NW
CHUNK = 80
NCHUNK = EPW // CHUNK
NROWC = N // CHUNK
SCALE = 1.0 / (T * np.sqrt(float(C)))

_f32 = jnp.float32


def _proj_body(x_ref, w_ref, y_ref):
    x = x_ref[...]
    for t in range(T):
        y_ref[t] = jnp.dot(x, w_ref[t], preferred_element_type=_f32) * SCALE


_BN = 2000


def _project(x, w3):
    return pl.pallas_call(
        _proj_body,
        grid=(N // _BN,),
        in_specs=[
            pl.BlockSpec((_BN, C), lambda nb: (nb, 0)),
            pl.BlockSpec((T, C, C), lambda nb: (0, 0, 0)),
        ],
        out_specs=pl.BlockSpec((T, _BN, C), lambda nb: (0, nb, 0)),
        out_shape=jax.ShapeDtypeStruct((T, N, C), _f32),
    )(x, w3)


def _gidx_body(ei_ref, typ_ref, g_ref, r_ref):
    g_ref[...] = typ_ref[...] * N + ei_ref[1]
    r_ref[...] = ei_ref[0]


def _edge_lists(edge_index, edge_type):
    e2 = E // C
    return pl.pallas_call(
        _gidx_body,
        out_shape=(
            jax.ShapeDtypeStruct((e2, C), jnp.int32),
            jax.ShapeDtypeStruct((e2, C), jnp.int32),
        ),
    )(edge_index.reshape(2, e2, C), edge_type.reshape(e2, C))


_mesh = plsc.VectorSubcoreMesh(core_axis_name="c", subcore_axis_name="s")


@functools.partial(
    pl.kernel,
    out_type=jax.ShapeDtypeStruct((NC, N, C), _f32),
    mesh=_mesh,
    scratch_types=[
        pltpu.VMEM((NCHUNK, CHUNK), jnp.int32),
        pltpu.VMEM((1, CHUNK), jnp.int32),
        pltpu.VMEM((1, CHUNK), jnp.int32),
        pltpu.VMEM((1, CHUNK), jnp.int32),
        pltpu.VMEM((CHUNK, C), _f32),
        pltpu.VMEM((CHUNK, C), _f32),
        pltpu.VMEM((CHUNK, C), _f32),
        pltpu.VMEM_SHARED((N, C), _f32),
        pltpu.SemaphoreType.DMA,
        pltpu.SemaphoreType.DMA,
        pltpu.SemaphoreType.DMA,
    ],
    compiler_params=pltpu.CompilerParams(use_tc_tiling_on_sc=False),
)
def _sc_aggregate(y_hbm, gidx_hbm, rows_hbm, out_hbm,
                  gidx_v, rb0, rb1, rb2, gb0, gb1, gb2, acc,
                  sem0, sem1, sem2):
    cid = lax.axis_index("c")
    sid = lax.axis_index("s")
    wid = sid * NC + cid
    rows_h = rows_hbm.at[wid]

    pltpu.sync_copy(gidx_hbm.at[wid], gidx_v)

    zeros16 = jnp.zeros((16,), _f32)

    def _zrow(i, carry):
        for j in range(C // 16):
            gb0[i, pl.ds(j * 16, 16)] = zeros16
        return carry

    lax.fori_loop(0, CHUNK, _zrow, 0)
    for i in range(pl.cdiv(NROWC, NS)):
        zc = i * NS + sid

        @pl.when(zc < NROWC)
        def _zero_chunk():
            off = pl.multiple_of(zc * CHUNK, CHUNK)
            pltpu.sync_copy(gb0, acc.at[pl.ds(off, CHUNK)])

    plsc.subcore_barrier()

    bufs = ((gb0, rb0, sem0), (gb1, rb1, sem1), (gb2, rb2, sem2))

    def _start(k, c):
        gb, rb, sem = bufs[k]
        pltpu.async_copy(y_hbm.at[gidx_v.at[c]], gb, sem)
        pltpu.async_copy(rows_h.at[pl.ds(c, 1)], rb, sem)

    def _finish(k, c):
        gb, rb, sem = bufs[k]
        pltpu.make_async_copy(y_hbm.at[gidx_v.at[c]], gb, sem).wait()
        pltpu.make_async_copy(rows_h.at[pl.ds(c, 1)], rb, sem).wait()
        pltpu.sync_copy(gb, acc.at[rb.at[0]], add=True)

    _start(0, 0)
    _start(1, 1)
    _start(2, 2)

    def _body(j, carry):
        c = 3 * j
        _finish(0, c)
        _start(0, c + 3)
        _finish(1, c + 1)
        _start(1, c + 4)
        _finish(2, c + 2)
        _start(2, c + 5)
        return carry

    lax.fori_loop(0, (NCHUNK - 5) // 3, _body, 0)
    _finish(0, NCHUNK - 5)
    _start(0, NCHUNK - 2)
    _finish(1, NCHUNK - 4)
    _start(1, NCHUNK - 1)
    _finish(2, NCHUNK - 3)
    _finish(0, NCHUNK - 2)
    _finish(1, NCHUNK - 1)

    plsc.subcore_barrier()
    for i in range(pl.cdiv(NROWC, NS)):
        wc = i * NS + sid

        @pl.when(wc < NROWC)
        def _wb_chunk():
            off = pl.multiple_of(wc * CHUNK, CHUNK)
            pltpu.sync_copy(acc.at[pl.ds(off, CHUNK)],
                            out_hbm.at[cid, pl.ds(off, CHUNK)])


def _add_body(p_ref, o_ref):
    o_ref[...] = p_ref[0] + p_ref[1]


def _final_add(partials):
    return pl.pallas_call(
        _add_body,
        grid=(N // _BN,),
        in_specs=[pl.BlockSpec((NC, _BN, C), lambda i: (0, i, 0))],
        out_specs=pl.BlockSpec((_BN, C), lambda i: (i, 0)),
        out_shape=jax.ShapeDtypeStruct((N, C), _f32),
    )(partials)


def kernel(x, edge_index, edge_type, W):
    w3 = W.reshape(T, C, C)
    y = _project(x, w3)
    g2d, r2d = _edge_lists(edge_index, edge_type)
    gidx3 = g2d.reshape(NW, NCHUNK, CHUNK)
    rows3 = r2d.reshape(NW, NCHUNK, CHUNK)
    partials = _sc_aggregate(y.reshape(T * N, C), gidx3, rows3)
    return _final_add(partials)


# speedup vs baseline: 15.0144x; 1.0123x over previous
import functools

import jax
import jax.numpy as jnp
import numpy as np
from jax import lax
from jax.experimental import pallas as pl
from jax.experimental.pallas import tpu as pltpu
from jax.experimental.pallas import tpu_sc as plsc

N = 10000
E = 320000
C = 128
T = 7
NC, NS = 2, 16
NW = NC * NS
EPW = E // NW
CHUNK = 80
NCHUNK = EPW // CHUNK
NROWC = N // CHUNK
SCALE = 1.0 / (T * np.sqrt(float(C)))

_f32 = jnp.float32


_BN = 2000
_E2 = E // C


def _proj_body(x_ref, w_ref, ei_ref, typ_ref, y_ref, g_ref, r_ref):
    x = x_ref[...]
    for t in range(T):
        y_ref[t] = jnp.dot(x, w_ref[t], preferred_element_type=_f32) * SCALE

    @pl.when(pl.program_id(0) == 0)
    def _edge_lists():
        g_ref[...] = typ_ref[...] * N + ei_ref[1]
        r_ref[...] = ei_ref[0]


def _project(x, w3, ei3, typ2d):
    zero3 = lambda nb: (0, 0, 0)
    zero2 = lambda nb: (0, 0)
    return pl.pallas_call(
        _proj_body,
        grid=(N // _BN,),
        in_specs=[
            pl.BlockSpec((_BN, C), lambda nb: (nb, 0)),
            pl.BlockSpec((T, C, C), zero3),
            pl.BlockSpec((2, _E2, C), zero3),
            pl.BlockSpec((_E2, C), zero2),
        ],
        out_specs=[
            pl.BlockSpec((T, _BN, C), lambda nb: (0, nb, 0)),
            pl.BlockSpec((_E2, C), zero2),
            pl.BlockSpec((_E2, C), zero2),
        ],
        out_shape=(
            jax.ShapeDtypeStruct((T, N, C), _f32),
            jax.ShapeDtypeStruct((_E2, C), jnp.int32),
            jax.ShapeDtypeStruct((_E2, C), jnp.int32),
        ),
    )(x, w3, ei3, typ2d)


_mesh = plsc.VectorSubcoreMesh(core_axis_name="c", subcore_axis_name="s")


@functools.partial(
    pl.kernel,
    out_type=jax.ShapeDtypeStruct((NC, N, C), _f32),
    mesh=_mesh,
    scratch_types=[
        pltpu.VMEM((NCHUNK, CHUNK), jnp.int32),
        pltpu.VMEM((1, CHUNK), jnp.int32),
        pltpu.VMEM((1, CHUNK), jnp.int32),
        pltpu.VMEM((1, CHUNK), jnp.int32),
        pltpu.VMEM((CHUNK, C), _f32),
        pltpu.VMEM((CHUNK, C), _f32),
        pltpu.VMEM((CHUNK, C), _f32),
        pltpu.VMEM_SHARED((N, C), _f32),
        pltpu.SemaphoreType.DMA,
        pltpu.SemaphoreType.DMA,
        pltpu.SemaphoreType.DMA,
    ],
    compiler_params=pltpu.CompilerParams(use_tc_tiling_on_sc=False),
)
def _sc_aggregate(y_hbm, gidx_hbm, rows_hbm, out_hbm,
                  gidx_v, rb0, rb1, rb2, gb0, gb1, gb2, acc,
                  sem0, sem1, sem2):
    cid = lax.axis_index("c")
    sid = lax.axis_index("s")
    wid = sid * NC + cid
    rows_h = rows_hbm.at[wid]

    pltpu.sync_copy(gidx_hbm.at[wid], gidx_v)

    zeros16 = jnp.zeros((16,), _f32)

    def _zrow(i, carry):
        for j in range(C // 16):
            gb0[i, pl.ds(j * 16, 16)] = zeros16
        return carry

    lax.fori_loop(0, CHUNK, _zrow, 0)
    for i in range(pl.cdiv(NROWC, NS)):
        zc = i * NS + sid

        @pl.when(zc < NROWC)
        def _zero_chunk():
            off = pl.multiple_of(zc * CHUNK, CHUNK)
            pltpu.sync_copy(gb0, acc.at[pl.ds(off, CHUNK)])

    plsc.subcore_barrier()

    bufs = ((gb0, rb0, sem0), (gb1, rb1, sem1), (gb2, rb2, sem2))

    def _start(k, c):
        gb, rb, sem = bufs[k]
        pltpu.async_copy(y_hbm.at[gidx_v.at[c]], gb, sem)
        pltpu.async_copy(rows_h.at[pl.ds(c, 1)], rb, sem)

    def _finish(k, c):
        gb, rb, sem = bufs[k]
        pltpu.make_async_copy(y_hbm.at[gidx_v.at[c]], gb, sem).wait()
        pltpu.make_async_copy(rows_h.at[pl.ds(c, 1)], rb, sem).wait()
        pltpu.sync_copy(gb, acc.at[rb.at[0]], add=True)

    _start(0, 0)
    _start(1, 1)
    _start(2, 2)

    def _body(j, carry):
        c = 3 * j
        _finish(0, c)
        _start(0, c + 3)
        _finish(1, c + 1)
        _start(1, c + 4)
        _finish(2, c + 2)
        _start(2, c + 5)
        return carry

    lax.fori_loop(0, (NCHUNK - 5) // 3, _body, 0)
    _finish(0, NCHUNK - 5)
    _start(0, NCHUNK - 2)
    _finish(1, NCHUNK - 4)
    _start(1, NCHUNK - 1)
    _finish(2, NCHUNK - 3)
    _finish(0, NCHUNK - 2)
    _finish(1, NCHUNK - 1)

    plsc.subcore_barrier()
    for i in range(pl.cdiv(NROWC, NS)):
        wc = i * NS + sid

        @pl.when(wc < NROWC)
        def _wb_chunk():
            off = pl.multiple_of(wc * CHUNK, CHUNK)
            pltpu.sync_copy(acc.at[pl.ds(off, CHUNK)],
                            out_hbm.at[cid, pl.ds(off, CHUNK)])


def _add_body(p_ref, o_ref):
    o_ref[...] = p_ref[0] + p_ref[1]


def _final_add(partials):
    return pl.pallas_call(
        _add_body,
        grid=(N // _BN,),
        in_specs=[pl.BlockSpec((NC, _BN, C), lambda i: (0, i, 0))],
        out_specs=pl.BlockSpec((_BN, C), lambda i: (i, 0)),
        out_shape=jax.ShapeDtypeStruct((N, C), _f32),
    )(partials)


def kernel(x, edge_index, edge_type, W):
    w3 = W.reshape(T, C, C)
    ei3 = edge_index.reshape(2, _E2, C)
    typ2d = edge_type.reshape(_E2, C)
    y, g2d, r2d = _project(x, w3, ei3, typ2d)
    gidx3 = g2d.reshape(NW, NCHUNK, CHUNK)
    rows3 = r2d.reshape(NW, NCHUNK, CHUNK)
    partials = _sc_aggregate(y.reshape(T * N, C), gidx3, rows3)
    return _final_add(partials)
